# baseline (speedup 1.0000x reference)
import functools

import jax
import jax.numpy as jnp
from jax import lax
from jax.experimental import pallas as pl
from jax.experimental.pallas import tpu as pltpu
from jax.experimental.pallas import tpu_sc as plsc

N = 10000
E = 320000
D_IN = 128
H = 128
D_OUT = 4

NP = 10240
PAD_NODE = N
NW = 32
CHUNK = 128
NBUF = 4
EPW = 10240
EP = EPW * NW
RPS = NP // 16
ROW_BLK = 256
GRID = NP // ROW_BLK

_mesh = plsc.VectorSubcoreMesh(core_axis_name="c", subcore_axis_name="s")
SLOWC = 1


@functools.partial(
    pl.kernel,
    out_type=jax.ShapeDtypeStruct((2, NP), jnp.float32),
    mesh=_mesh,
    scratch_types=[
        pltpu.VMEM((EPW // CHUNK, CHUNK), jnp.int32),
        pltpu.VMEM((CHUNK,), jnp.float32),
        pltpu.VMEM((RPS,), jnp.float32),
        pltpu.VMEM_SHARED((NP,), jnp.float32),
    ],
)
def _sc_degree(dst_hbm, out_hbm, idx_v, ones_v, zeros_v, deg_sh):
    c = lax.axis_index("c")
    s = lax.axis_index("s")
    w = c * 16 + s
    nk = EPW // CHUNK
    for i in range(CHUNK // 16):
        ones_v[pl.ds(i * 16, 16)] = jnp.ones((16,), jnp.float32)
    for i in range(RPS // 16):
        zeros_v[pl.ds(i * 16, 16)] = jnp.zeros((16,), jnp.float32)
    pltpu.sync_copy(dst_hbm.at[pl.ds(w * nk, nk)], idx_v)
    pltpu.sync_copy(zeros_v, deg_sh.at[pl.ds(s * RPS, RPS)])
    plsc.subcore_barrier()

    def chunk(k, carry):
        pltpu.sync_copy(ones_v, deg_sh.at[idx_v.at[k]], add=True)
        return carry

    lax.fori_loop(0, nk, chunk, 0)
    plsc.subcore_barrier()
    pltpu.sync_copy(deg_sh.at[pl.ds(s * RPS, RPS)],
                    out_hbm.at[c, pl.ds(s * RPS, RPS)])


def _make_sc_prop(d):
    params = None if d % 128 == 0 else pltpu.CompilerParams(use_tc_tiling_on_sc=False)
    chunk = CHUNK
    if d >= 128:
        nbuf, nks, phs, nkf, phf, kpm = 2, 64, 2, 96, 3, 32
    else:
        nbuf, nks, phs, nkf, phf, kpm = 4, 56, 1, 104, 1, 104
    zr = 32

    @functools.partial(
        pl.kernel,
        out_type=jax.ShapeDtypeStruct((2, NP, d), jnp.float32),
        mesh=_mesh,
        compiler_params=params,
        scratch_types=[
            pltpu.VMEM((kpm, chunk), jnp.int32),
            pltpu.VMEM((kpm, chunk), jnp.int32),
            pltpu.VMEM((nbuf, chunk, d), jnp.float32),
            pltpu.VMEM((zr, d), jnp.float32),
            pltpu.VMEM_SHARED((NP, d), jnp.float32),
            pltpu.SemaphoreType.DMA((nbuf,)),
            pltpu.SemaphoreType.DMA((nbuf,)),
        ],
    )
    def prop(u_hbm, src_hbm, dst_hbm, out_hbm, src_v, dst_v, rows_v, zb,
             acc_sh, gsem, ssem):
        c = lax.axis_index("c")
        s = lax.axis_index("s")
        for i in range(zr):
            for q in range(d // 16):
                zb[i, pl.ds(q * 16, 16)] = jnp.zeros((16,), jnp.float32)
        for t in range(RPS // zr):
            pltpu.sync_copy(zb, acc_sh.at[pl.ds(s * RPS + t * zr, zr)])
        plsc.subcore_barrier()

        def run_edges(base, nch, phn):
            kp = nch // phn
            for p in range(phn):
                pltpu.sync_copy(src_hbm.at[pl.ds(base + p * kp, kp)],
                                src_v.at[pl.ds(0, kp)])
                pltpu.sync_copy(dst_hbm.at[pl.ds(base + p * kp, kp)],
                                dst_v.at[pl.ds(0, kp)])
                for b in range(nbuf):
                    pltpu.async_copy(u_hbm.at[src_v.at[b]], rows_v.at[b],
                                     gsem.at[b])

                def group(g, carry):
                    for b in range(nbuf):
                        j = g * nbuf + b
                        b2 = (b + 2) % nbuf
                        pltpu.make_async_copy(u_hbm.at[src_v.at[j]],
                                              rows_v.at[b], gsem.at[b]).wait()
                        if nbuf == 2:
                            pltpu.sync_copy(rows_v.at[b], acc_sh.at[dst_v.at[j]],
                                            add=True)

                            @pl.when(j + 2 < kp)
                            def _():
                                pltpu.async_copy(u_hbm.at[src_v.at[j + 2]],
                                                 rows_v.at[b], gsem.at[b])
                        else:
                            pltpu.async_copy(rows_v.at[b],
                                             acc_sh.at[dst_v.at[j]],
                                             ssem.at[b], add=True, priority=1)

                            @pl.when((j >= 2) & (j + 2 < kp))
                            def _():
                                pltpu.make_async_copy(rows_v.at[b2],
                                                      acc_sh.at[dst_v.at[j]],
                                                      ssem.at[b2]).wait()
                                pltpu.async_copy(u_hbm.at[src_v.at[j + 2]],
                                                 rows_v.at[b2], gsem.at[b2])
                    return carry

                lax.fori_loop(0, kp // nbuf, group, 0)
                if nbuf != 2:
                    for b in range(nbuf):
                        pltpu.make_async_copy(rows_v.at[b],
                                              acc_sh.at[dst_v.at[0]],
                                              ssem.at[b]).wait()

        @pl.when(c == SLOWC)
        def _():
            run_edges(s * nks, nks, phs)

        @pl.when(c != SLOWC)
        def _():
            run_edges(16 * nks + s * nkf, nkf, phf)

        plsc.subcore_barrier()
        pltpu.sync_copy(acc_sh.at[pl.ds(s * RPS, RPS)],
                        out_hbm.at[c, pl.ds(s * RPS, RPS)])

    return prop


_sc_prop_h = _make_sc_prop(H)
_sc_prop_16 = _make_sc_prop(16)


def _tc_first_body(x_ref, w_ref, d0_ref, d1_ref, u_ref, dis_ref):
    deg = d0_ref[...] + d1_ref[...] + 1.0
    dis = lax.rsqrt(deg)
    u_ref[...] = jnp.dot(x_ref[...], w_ref[...],
                         preferred_element_type=jnp.float32) * dis
    dis_ref[...] = dis


def _tc_first(xp, w1, degp):
    deg2 = degp.reshape(2 * NP, 1)
    return pl.pallas_call(
        _tc_first_body,
        grid=(GRID,),
        in_specs=[
            pl.BlockSpec((ROW_BLK, D_IN), lambda i: (i, 0)),
            pl.BlockSpec((D_IN, H), lambda i: (0, 0)),
            pl.BlockSpec((ROW_BLK, 1), lambda i: (i, 0)),
            pl.BlockSpec((ROW_BLK, 1), lambda i: (i + GRID, 0)),
        ],
        out_specs=[
            pl.BlockSpec((ROW_BLK, H), lambda i: (i, 0)),
            pl.BlockSpec((ROW_BLK, 1), lambda i: (i, 0)),
        ],
        out_shape=[
            jax.ShapeDtypeStruct((NP, H), jnp.float32),
            jax.ShapeDtypeStruct((NP, 1), jnp.float32),
        ],
    )(xp, w1, deg2, deg2)


def _tc_mid_body(v0_ref, v1_ref, u_ref, dis_ref, b_ref, w_ref, out_ref):
    dis = dis_ref[...]
    z = dis * (v0_ref[0] + v1_ref[0] + u_ref[...]) + b_ref[...]
    z = jnp.where(z >= 0, z, 0.2 * z)
    out_ref[...] = jnp.dot(z, w_ref[...],
                           preferred_element_type=jnp.float32) * dis


def _tc_mid(v, u, dis, b, w):
    d = u.shape[1]
    dout = w.shape[1]
    return pl.pallas_call(
        _tc_mid_body,
        grid=(GRID,),
        in_specs=[
            pl.BlockSpec((1, ROW_BLK, d), lambda i: (0, i, 0)),
            pl.BlockSpec((1, ROW_BLK, d), lambda i: (1, i, 0)),
            pl.BlockSpec((ROW_BLK, d), lambda i: (i, 0)),
            pl.BlockSpec((ROW_BLK, 1), lambda i: (i, 0)),
            pl.BlockSpec((1, d), lambda i: (0, 0)),
            pl.BlockSpec((d, dout), lambda i: (0, 0)),
        ],
        out_specs=pl.BlockSpec((ROW_BLK, dout), lambda i: (i, 0)),
        out_shape=jax.ShapeDtypeStruct((NP, dout), jnp.float32),
    )(v, v, u, dis, b, w)


def _tc_final_body(v0_ref, v1_ref, u_ref, dis_ref, b_ref, out_ref):
    out_ref[...] = dis_ref[...] * (v0_ref[0] + v1_ref[0] + u_ref[...]) + b_ref[...]


def _tc_final(v, u, dis, b):
    d = u.shape[1]
    return pl.pallas_call(
        _tc_final_body,
        grid=(GRID,),
        in_specs=[
            pl.BlockSpec((1, ROW_BLK, d), lambda i: (0, i, 0)),
            pl.BlockSpec((1, ROW_BLK, d), lambda i: (1, i, 0)),
            pl.BlockSpec((ROW_BLK, d), lambda i: (i, 0)),
            pl.BlockSpec((ROW_BLK, 1), lambda i: (i, 0)),
            pl.BlockSpec((1, d), lambda i: (0, 0)),
        ],
        out_specs=pl.BlockSpec((ROW_BLK, d), lambda i: (i, 0)),
        out_shape=jax.ShapeDtypeStruct((NP, d), jnp.float32),
    )(v, v, u, dis, b)


def kernel(x, edge_index, W1, b1, W2, b2, W3, b3):
    src = edge_index[0]
    dst = edge_index[1]
    pad_e = jnp.full((EP - E,), PAD_NODE, jnp.int32)
    srcp = jnp.concatenate([src, pad_e]).reshape(EP // CHUNK, CHUNK)
    dstp = jnp.concatenate([dst, pad_e]).reshape(EP // CHUNK, CHUNK)
    xp = jnp.pad(x, ((0, NP - N), (0, 0)))
    w3p = jnp.pad(W3, ((0, 0), (0, 16 - D_OUT)))
    b3p = jnp.pad(b3, (0, 16 - D_OUT))

    degp = _sc_degree(dstp)
    u1, dis = _tc_first(xp, W1, degp)
    v1 = _sc_prop_h(u1, srcp, dstp)
    u2 = _tc_mid(v1, u1, dis, b1.reshape(1, H), W2)
    v2 = _sc_prop_h(u2, srcp, dstp)
    u3 = _tc_mid(v2, u2, dis, b2.reshape(1, H), w3p)
    v3 = _sc_prop_16(u3, srcp, dstp)
    outp = _tc_final(v3, u3, dis, b3p.reshape(1, 16))
    return outp[:N, :D_OUT]


# speedup vs baseline: 1.0121x; 1.0121x over previous
import functools

import jax
import jax.numpy as jnp
from jax import lax
from jax.experimental import pallas as pl
from jax.experimental.pallas import tpu as pltpu
from jax.experimental.pallas import tpu_sc as plsc

N = 10000
E = 320000
D_IN = 128
H = 128
D_OUT = 4

NP = 10240
PAD_NODE = N
NW = 32
CHUNK = 128
NBUF = 4
EPW = 10240
EP = EPW * NW
RPS = NP // 16
ROW_BLK = 256
GRID = NP // ROW_BLK

_mesh = plsc.VectorSubcoreMesh(core_axis_name="c", subcore_axis_name="s")
SLOWC = 0


@functools.partial(
    pl.kernel,
    out_type=jax.ShapeDtypeStruct((2, NP), jnp.float32),
    mesh=_mesh,
    scratch_types=[
        pltpu.VMEM((EPW // CHUNK, CHUNK), jnp.int32),
        pltpu.VMEM((CHUNK,), jnp.float32),
        pltpu.VMEM((RPS,), jnp.float32),
        pltpu.VMEM_SHARED((NP,), jnp.float32),
    ],
)
def _sc_degree(dst_hbm, out_hbm, idx_v, ones_v, zeros_v, deg_sh):
    c = lax.axis_index("c")
    s = lax.axis_index("s")
    w = c * 16 + s
    nk = EPW // CHUNK
    for i in range(CHUNK // 16):
        ones_v[pl.ds(i * 16, 16)] = jnp.ones((16,), jnp.float32)
    for i in range(RPS // 16):
        zeros_v[pl.ds(i * 16, 16)] = jnp.zeros((16,), jnp.float32)
    pltpu.sync_copy(dst_hbm.at[pl.ds(w * nk, nk)], idx_v)
    pltpu.sync_copy(zeros_v, deg_sh.at[pl.ds(s * RPS, RPS)])
    plsc.subcore_barrier()

    def chunk(k, carry):
        pltpu.sync_copy(ones_v, deg_sh.at[idx_v.at[k]], add=True)
        return carry

    lax.fori_loop(0, nk, chunk, 0)
    plsc.subcore_barrier()
    pltpu.sync_copy(deg_sh.at[pl.ds(s * RPS, RPS)],
                    out_hbm.at[c, pl.ds(s * RPS, RPS)])


def _make_sc_prop(d):
    params = None if d % 128 == 0 else pltpu.CompilerParams(use_tc_tiling_on_sc=False)
    chunk = CHUNK
    if d >= 128:
        nbuf, nks, phs, nkf, phf, kpm = 2, 80, 2, 80, 2, 40
    else:
        nbuf, nks, phs, nkf, phf, kpm = 4, 80, 1, 80, 1, 80
    zr = 32

    @functools.partial(
        pl.kernel,
        out_type=jax.ShapeDtypeStruct((2, NP, d), jnp.float32),
        mesh=_mesh,
        compiler_params=params,
        scratch_types=[
            pltpu.VMEM((kpm, chunk), jnp.int32),
            pltpu.VMEM((kpm, chunk), jnp.int32),
            pltpu.VMEM((nbuf, chunk, d), jnp.float32),
            pltpu.VMEM((zr, d), jnp.float32),
            pltpu.VMEM_SHARED((NP, d), jnp.float32),
            pltpu.SemaphoreType.DMA((nbuf,)),
            pltpu.SemaphoreType.DMA((nbuf,)),
        ],
    )
    def prop(u_hbm, src_hbm, dst_hbm, out_hbm, src_v, dst_v, rows_v, zb,
             acc_sh, gsem, ssem):
        c = lax.axis_index("c")
        s = lax.axis_index("s")
        for i in range(zr):
            for q in range(d // 16):
                zb[i, pl.ds(q * 16, 16)] = jnp.zeros((16,), jnp.float32)
        for t in range(RPS // zr):
            pltpu.sync_copy(zb, acc_sh.at[pl.ds(s * RPS + t * zr, zr)])
        plsc.subcore_barrier()

        def run_edges(base, nch, phn):
            kp = nch // phn
            for p in range(phn):
                pltpu.sync_copy(src_hbm.at[pl.ds(base + p * kp, kp)],
                                src_v.at[pl.ds(0, kp)])
                pltpu.sync_copy(dst_hbm.at[pl.ds(base + p * kp, kp)],
                                dst_v.at[pl.ds(0, kp)])
                for b in range(nbuf):
                    pltpu.async_copy(u_hbm.at[src_v.at[b]], rows_v.at[b],
                                     gsem.at[b])

                def group(g, carry):
                    for b in range(nbuf):
                        j = g * nbuf + b
                        b2 = (b + 2) % nbuf
                        pltpu.make_async_copy(u_hbm.at[src_v.at[j]],
                                              rows_v.at[b], gsem.at[b]).wait()
                        if nbuf == 2:
                            pltpu.sync_copy(rows_v.at[b], acc_sh.at[dst_v.at[j]],
                                            add=True)

                            @pl.when(j + 2 < kp)
                            def _():
                                pltpu.async_copy(u_hbm.at[src_v.at[j + 2]],
                                                 rows_v.at[b], gsem.at[b])
                        else:
                            pltpu.async_copy(rows_v.at[b],
                                             acc_sh.at[dst_v.at[j]],
                                             ssem.at[b], add=True, priority=1)

                            @pl.when((j >= 2) & (j + 2 < kp))
                            def _():
                                pltpu.make_async_copy(rows_v.at[b2],
                                                      acc_sh.at[dst_v.at[j]],
                                                      ssem.at[b2]).wait()
                                pltpu.async_copy(u_hbm.at[src_v.at[j + 2]],
                                                 rows_v.at[b2], gsem.at[b2])
                    return carry

                lax.fori_loop(0, kp // nbuf, group, 0)
                if nbuf != 2:
                    for b in range(nbuf):
                        pltpu.make_async_copy(rows_v.at[b],
                                              acc_sh.at[dst_v.at[0]],
                                              ssem.at[b]).wait()

        @pl.when(c == SLOWC)
        def _():
            run_edges(s * nks, nks, phs)

        @pl.when(c != SLOWC)
        def _():
            run_edges(16 * nks + s * nkf, nkf, phf)

        plsc.subcore_barrier()
        pltpu.sync_copy(acc_sh.at[pl.ds(s * RPS, RPS)],
                        out_hbm.at[c, pl.ds(s * RPS, RPS)])

    return prop


_sc_prop_h = _make_sc_prop(H)
_sc_prop_16 = _make_sc_prop(16)


def _tc_first_body(x_ref, w_ref, d0_ref, d1_ref, u_ref, dis_ref):
    deg = d0_ref[...] + d1_ref[...] + 1.0
    dis = lax.rsqrt(deg)
    u_ref[...] = jnp.dot(x_ref[...], w_ref[...],
                         preferred_element_type=jnp.float32) * dis
    dis_ref[...] = dis


def _tc_first(xp, w1, degp):
    deg2 = degp.reshape(2 * NP, 1)
    return pl.pallas_call(
        _tc_first_body,
        grid=(GRID,),
        in_specs=[
            pl.BlockSpec((ROW_BLK, D_IN), lambda i: (i, 0)),
            pl.BlockSpec((D_IN, H), lambda i: (0, 0)),
            pl.BlockSpec((ROW_BLK, 1), lambda i: (i, 0)),
            pl.BlockSpec((ROW_BLK, 1), lambda i: (i + GRID, 0)),
        ],
        out_specs=[
            pl.BlockSpec((ROW_BLK, H), lambda i: (i, 0)),
            pl.BlockSpec((ROW_BLK, 1), lambda i: (i, 0)),
        ],
        out_shape=[
            jax.ShapeDtypeStruct((NP, H), jnp.float32),
            jax.ShapeDtypeStruct((NP, 1), jnp.float32),
        ],
    )(xp, w1, deg2, deg2)


def _tc_mid_body(v0_ref, v1_ref, u_ref, dis_ref, b_ref, w_ref, out_ref):
    dis = dis_ref[...]
    z = dis * (v0_ref[0] + v1_ref[0] + u_ref[...]) + b_ref[...]
    z = jnp.where(z >= 0, z, 0.2 * z)
    out_ref[...] = jnp.dot(z, w_ref[...],
                           preferred_element_type=jnp.float32) * dis


def _tc_mid(v, u, dis, b, w):
    d = u.shape[1]
    dout = w.shape[1]
    return pl.pallas_call(
        _tc_mid_body,
        grid=(GRID,),
        in_specs=[
            pl.BlockSpec((1, ROW_BLK, d), lambda i: (0, i, 0)),
            pl.BlockSpec((1, ROW_BLK, d), lambda i: (1, i, 0)),
            pl.BlockSpec((ROW_BLK, d), lambda i: (i, 0)),
            pl.BlockSpec((ROW_BLK, 1), lambda i: (i, 0)),
            pl.BlockSpec((1, d), lambda i: (0, 0)),
            pl.BlockSpec((d, dout), lambda i: (0, 0)),
        ],
        out_specs=pl.BlockSpec((ROW_BLK, dout), lambda i: (i, 0)),
        out_shape=jax.ShapeDtypeStruct((NP, dout), jnp.float32),
    )(v, v, u, dis, b, w)


def _tc_final_body(v0_ref, v1_ref, u_ref, dis_ref, b_ref, out_ref):
    out_ref[...] = dis_ref[...] * (v0_ref[0] + v1_ref[0] + u_ref[...]) + b_ref[...]


def _tc_final(v, u, dis, b):
    d = u.shape[1]
    return pl.pallas_call(
        _tc_final_body,
        grid=(GRID,),
        in_specs=[
            pl.BlockSpec((1, ROW_BLK, d), lambda i: (0, i, 0)),
            pl.BlockSpec((1, ROW_BLK, d), lambda i: (1, i, 0)),
            pl.BlockSpec((ROW_BLK, d), lambda i: (i, 0)),
            pl.BlockSpec((ROW_BLK, 1), lambda i: (i, 0)),
            pl.BlockSpec((1, d), lambda i: (0, 0)),
        ],
        out_specs=pl.BlockSpec((ROW_BLK, d), lambda i: (i, 0)),
        out_shape=jax.ShapeDtypeStruct((NP, d), jnp.float32),
    )(v, v, u, dis, b)


def kernel(x, edge_index, W1, b1, W2, b2, W3, b3):
    src = edge_index[0]
    dst = edge_index[1]
    pad_e = jnp.full((EP - E,), PAD_NODE, jnp.int32)
    srcp = jnp.concatenate([src, pad_e]).reshape(EP // CHUNK, CHUNK)
    dstp = jnp.concatenate([dst, pad_e]).reshape(EP // CHUNK, CHUNK)
    xp = jnp.pad(x, ((0, NP - N), (0, 0)))
    w3p = jnp.pad(W3, ((0, 0), (0, 16 - D_OUT)))
    b3p = jnp.pad(b3, (0, 16 - D_OUT))

    degp = _sc_degree(dstp)
    u1, dis = _tc_first(xp, W1, degp)
    v1 = _sc_prop_h(u1, srcp, dstp)
    u2 = _tc_mid(v1, u1, dis, b1.reshape(1, H), W2)
    v2 = _sc_prop_h(u2, srcp, dstp)
    u3 = _tc_mid(v2, u2, dis, b2.reshape(1, H), w3p)
    v3 = _sc_prop_16(u3, srcp, dstp)
    outp = _tc_final(v3, u3, dis, b3p.reshape(1, 16))
    return outp[:N, :D_OUT]
